# baseline (speedup 1.0000x reference)
import jax
import jax.numpy as jnp
from jax.experimental import pallas as pl
from jax.experimental.pallas import tpu as pltpu

_ROWS = 2048
_COLS = 8192
_BLOCK_COLS = 256
_NBLK = _COLS // _BLOCK_COLS
_NEG = float("-inf")


def _shift_up(a, k):
    return jnp.concatenate(
        [a[k:, :], jnp.full((k, a.shape[1]), _NEG, a.dtype)], axis=0
    )


def _sliding_max45(x):
    rows, cols = x.shape
    xp = jnp.concatenate([jnp.full((24, cols), _NEG, x.dtype), x], axis=0)
    f = xp
    for k in (1, 2, 4, 8, 16):
        f = jnp.maximum(f, _shift_up(f, k))
    f45 = jnp.maximum(f, _shift_up(f, 13))
    return f45[2 : rows + 2, :]


def _fused_kernel(x_ref, o_ref, park_ref, gmax_ref):
    i = pl.program_id(1)

    @pl.when(pl.program_id(0) == 0)
    def _nms_phase():
        x = x_ref[...]
        m = _sliding_max45(x)
        filt = jnp.where(x == m, x, jnp.float32(0.0))
        park_ref[:, pl.ds(i * _BLOCK_COLS, _BLOCK_COLS)] = filt.astype(
            jnp.bfloat16
        )
        bm = jnp.max(x)

        @pl.when(i == 0)
        def _():
            gmax_ref[0] = bm

        @pl.when(i != 0)
        def _():
            gmax_ref[0] = jnp.maximum(gmax_ref[0], bm)

    @pl.when(pl.program_id(0) == 1)
    def _threshold_phase():
        thresh = gmax_ref[0] * jnp.float32(0.5)
        f = park_ref[:, pl.ds(i * _BLOCK_COLS, _BLOCK_COLS)].astype(
            jnp.float32
        )
        o_ref[...] = jnp.where(f >= thresh, f, jnp.float32(0.0))


@jax.jit
def kernel(preds):
    return pl.pallas_call(
        _fused_kernel,
        grid=(2, _NBLK),
        in_specs=[
            pl.BlockSpec(
                (_ROWS, _BLOCK_COLS),
                lambda p, i: (0, i * (1 - p) + (_NBLK - 1) * p),
            ),
        ],
        out_specs=pl.BlockSpec(
            (_ROWS, _BLOCK_COLS),
            lambda p, i: (0, i * p),
        ),
        out_shape=jax.ShapeDtypeStruct((_ROWS, _COLS), jnp.float32),
        scratch_shapes=[
            pltpu.VMEM((_ROWS, _COLS), jnp.bfloat16),
            pltpu.SMEM((1,), jnp.float32),
        ],
        compiler_params=pltpu.CompilerParams(
            dimension_semantics=("arbitrary", "arbitrary"),
        ),
    )(preds)


# speedup vs baseline: 1.3559x; 1.3559x over previous
import jax
import jax.numpy as jnp
from jax.experimental import pallas as pl
from jax.experimental.pallas import tpu as pltpu

_ROWS = 2048
_COLS = 8192
_BLOCK_COLS = 1024
_NBLK = _COLS // _BLOCK_COLS
_NEG = float("-inf")


def _shift_up(a, k):
    return jnp.concatenate(
        [a[k:, :], jnp.full((k, a.shape[1]), _NEG, a.dtype)], axis=0
    )


def _sliding_max45(x):
    rows, cols = x.shape
    xp = jnp.concatenate([jnp.full((24, cols), _NEG, x.dtype), x], axis=0)
    f = xp
    for k in (1, 2, 4, 8, 16):
        f = jnp.maximum(f, _shift_up(f, k))
    f45 = jnp.maximum(f, _shift_up(f, 13))
    return f45[2 : rows + 2, :]


def _fused_kernel(x_ref, o_ref, gmax_ref):
    i = pl.program_id(1)

    @pl.when(pl.program_id(0) == 0)
    def _max_phase():
        bm = jnp.max(x_ref[...])

        @pl.when(i == 0)
        def _():
            gmax_ref[0] = bm

        @pl.when(i != 0)
        def _():
            gmax_ref[0] = jnp.maximum(gmax_ref[0], bm)

    @pl.when(pl.program_id(0) == 1)
    def _nms_phase():
        thresh = gmax_ref[0] * jnp.float32(0.5)
        x = x_ref[...]
        m = _sliding_max45(x)
        keep = (x == m) & (x >= thresh)
        o_ref[...] = jnp.where(keep, x, jnp.float32(0.0))


@jax.jit
def kernel(preds):
    return pl.pallas_call(
        _fused_kernel,
        grid=(2, _NBLK),
        in_specs=[
            pl.BlockSpec((_ROWS, _BLOCK_COLS), lambda p, i: (0, i)),
        ],
        out_specs=pl.BlockSpec(
            (_ROWS, _BLOCK_COLS),
            lambda p, i: (0, i * p),
        ),
        out_shape=jax.ShapeDtypeStruct((_ROWS, _COLS), jnp.float32),
        scratch_shapes=[
            pltpu.SMEM((1,), jnp.float32),
        ],
        compiler_params=pltpu.CompilerParams(
            dimension_semantics=("arbitrary", "arbitrary"),
        ),
    )(preds)
